# baseline (speedup 1.0000x reference)
import functools

import jax
import jax.numpy as jnp
from jax.experimental import pallas as pl
from jax.experimental.pallas import tpu as pltpu
from jax.experimental.pallas import tpu_sc as plsc

VOCAB = 1000000
DIM = 64
BATCH = 16384
HIST = 20
NUM_IDX = BATCH * HIST

WINDOW = 128
GRID = NUM_IDX // WINDOW

_mesh = plsc.VectorSubcoreMesh(core_axis_name="core", subcore_axis_name="subcore")


@jax.jit
def _gather(table, idx):
    @functools.partial(
        pl.kernel,
        out_type=jax.ShapeDtypeStruct((NUM_IDX, DIM), table.dtype),
        mesh=_mesh,
        compiler_params=pltpu.CompilerParams(use_tc_tiling_on_sc=False),
    )
    def kern(table_hbm, idx_hbm, out_hbm):
        def body(i_vmem, o_vmem):
            pltpu.sync_copy(table_hbm.at[i_vmem.at[0]], o_vmem)

        pltpu.emit_pipeline(
            body,
            grid=(GRID,),
            in_specs=[pl.BlockSpec((1, WINDOW), index_map=lambda i: (0, i))],
            out_specs=[pl.BlockSpec((WINDOW, DIM), index_map=lambda i: (i, 0))],
            core_axis_name=("core", "subcore"),
            dimension_semantics=(pltpu.PARALLEL,),
        )(idx_hbm, out_hbm)

    return kern(table, idx)


def kernel(inputs, table):
    idx = inputs.reshape(1, NUM_IDX)
    out = _gather(table, idx)
    return out.reshape(BATCH, HIST, DIM)


# speedup vs baseline: 1.0405x; 1.0405x over previous
import functools

import jax
import jax.numpy as jnp
from jax.experimental import pallas as pl
from jax.experimental.pallas import tpu as pltpu
from jax.experimental.pallas import tpu_sc as plsc

VOCAB = 1000000
DIM = 64
BATCH = 16384
HIST = 20
NUM_IDX = BATCH * HIST

WINDOW = 128
K = 4
NROWS = NUM_IDX // WINDOW
GRID = NROWS // K

_mesh = plsc.VectorSubcoreMesh(core_axis_name="core", subcore_axis_name="subcore")


@jax.jit
def _gather(table, idx):
    @functools.partial(
        pl.kernel,
        out_type=jax.ShapeDtypeStruct((NUM_IDX, DIM), table.dtype),
        mesh=_mesh,
        scratch_types=[pltpu.SemaphoreType.DMA],
        compiler_params=pltpu.CompilerParams(use_tc_tiling_on_sc=False),
    )
    def kern(table_hbm, idx_hbm, out_hbm, sem):
        def body(i_vmem, o_vmem):
            copies = [
                pltpu.async_copy(
                    table_hbm.at[i_vmem.at[j]],
                    o_vmem.at[pl.ds(j * WINDOW, WINDOW)],
                    sem,
                )
                for j in range(K)
            ]
            for c in copies:
                c.wait()

        pltpu.emit_pipeline(
            body,
            grid=(GRID,),
            in_specs=[pl.BlockSpec((K, WINDOW), index_map=lambda i: (i, 0))],
            out_specs=[pl.BlockSpec((K * WINDOW, DIM), index_map=lambda i: (i, 0))],
            core_axis_name=("core", "subcore"),
            dimension_semantics=(pltpu.PARALLEL,),
        )(idx_hbm, out_hbm)

    return kern(table, idx)


def kernel(inputs, table):
    idx = inputs.reshape(NROWS, WINDOW)
    out = _gather(table, idx)
    return out.reshape(BATCH, HIST, DIM)
